# baseline (speedup 1.0000x reference)
import functools

import jax
import jax.numpy as jnp
from jax import lax
from jax.experimental import pallas as pl
from jax.experimental.pallas import tpu as pltpu
from jax.experimental.pallas import tpu_sc as plsc

NUM_ROWS = 100000
DIM = 64
RANK = 8

NC = 2
NS = 16
NW = NC * NS
IDX_TOTAL = 204800
PER_W = IDX_TOTAL // NW
BLK_I = 128
STREAMS_PER_GROUP = 5
GROUP = STREAMS_PER_GROUP * BLK_I
NGROUPS = PER_W // GROUP

FUSE_BLK = 512


def _fuse_body(wt_ref, bt_ref, a_ref, m0_ref, m1_ref, t2_ref):
    tt = wt_ref[...] + jnp.dot(a_ref[...].T, bt_ref[...],
                               preferred_element_type=jnp.float32)
    dn = (((0,), (1,)), ((), ()))
    even = lax.dot_general(m0_ref[...], tt, dn,
                           preferred_element_type=jnp.float32)
    odd = lax.dot_general(m1_ref[...], tt, dn,
                          preferred_element_type=jnp.float32)
    t2_ref[...] = jnp.concatenate([even, odd], axis=1)


def _fuse_table(Wt, A, Bt):
    r = lax.broadcasted_iota(jnp.int32, (FUSE_BLK, FUSE_BLK // 2), 0)
    p = lax.broadcasted_iota(jnp.int32, (FUSE_BLK, FUSE_BLK // 2), 1)
    m0 = (r == 2 * p).astype(jnp.float32)
    m1 = (r == 2 * p + 1).astype(jnp.float32)
    grid = ((NUM_ROWS + FUSE_BLK - 1) // FUSE_BLK,)
    return pl.pallas_call(
        _fuse_body,
        grid=grid,
        in_specs=[
            pl.BlockSpec((DIM, FUSE_BLK), lambda i: (0, i)),
            pl.BlockSpec((RANK, FUSE_BLK), lambda i: (0, i)),
            pl.BlockSpec((RANK, DIM), lambda i: (0, 0)),
            pl.BlockSpec((FUSE_BLK, FUSE_BLK // 2), lambda i: (0, 0)),
            pl.BlockSpec((FUSE_BLK, FUSE_BLK // 2), lambda i: (0, 0)),
        ],
        out_specs=pl.BlockSpec((FUSE_BLK // 2, 2 * DIM), lambda i: (i, 0)),
        out_shape=jax.ShapeDtypeStruct((NUM_ROWS // 2, 2 * DIM), jnp.float32),
    )(Wt, Bt, A, m0, m1)


def _sc_gather(table, idx3):
    mesh = plsc.VectorSubcoreMesh(core_axis_name="c", subcore_axis_name="s")

    @functools.partial(
        pl.kernel,
        mesh=mesh,
        compiler_params=pltpu.CompilerParams(use_tc_tiling_on_sc=False),
        out_type=jax.ShapeDtypeStruct(
            (NW, NGROUPS, STREAMS_PER_GROUP, BLK_I, DIM), jnp.float32),
        scratch_types=[
            pltpu.VMEM((NGROUPS * STREAMS_PER_GROUP, BLK_I), jnp.int32),
            pltpu.VMEM((STREAMS_PER_GROUP, BLK_I, DIM), jnp.float32),
            pltpu.SemaphoreType.DMA,
        ],
    )
    def k(table_hbm, idx_hbm, out_hbm, idx_v, rows_v, gsem):
        wid = lax.axis_index("s") * NC + lax.axis_index("c")
        pltpu.sync_copy(idx_hbm.at[wid], idx_v)

        def body(g, carry):
            descs = []
            for s in range(STREAMS_PER_GROUP):
                descs.append(pltpu.async_copy(
                    table_hbm.at[idx_v.at[g * STREAMS_PER_GROUP + s]],
                    rows_v.at[s], gsem))
            for d in descs:
                d.wait()
            pltpu.sync_copy(rows_v, out_hbm.at[wid, g])
            return carry

        lax.fori_loop(0, NGROUPS, body, 0)

    return k(table, idx3)


def _xpose_body(g_ref, z_ref):
    g = g_ref[0]
    zl = jnp.transpose(g[:, :DIM], (1, 0))
    zr = jnp.transpose(g[:, DIM:], (1, 0))
    z_ref[0] = jnp.concatenate([zl, zr], axis=1)


def _xpose(gp):
    return pl.pallas_call(
        _xpose_body,
        grid=(50, 2),
        in_specs=[pl.BlockSpec((1, 1024, 128), lambda j, kb: (j, kb, 0))],
        out_specs=pl.BlockSpec((1, DIM, 2048), lambda j, kb: (j, 0, kb)),
        out_shape=jax.ShapeDtypeStruct((50, DIM, 4096), jnp.float32),
    )(gp)


def kernel(x, W, A, B):
    t2 = _fuse_table(W.T, A, B.T)
    table = t2.reshape(NUM_ROWS, DIM)
    idx = jnp.transpose(x.T.reshape(50, 2, 2, 1024), (0, 1, 3, 2))
    idx3 = idx.reshape(NW, PER_W // BLK_I, BLK_I)
    out = _sc_gather(table, idx3)
    gp = out.reshape(50, 2048, 2 * DIM)
    z = _xpose(gp)
    return z.transpose(2, 0, 1)


# speedup vs baseline: 1.3469x; 1.3469x over previous
import functools

import jax
import jax.numpy as jnp
from jax import lax
from jax.experimental import pallas as pl
from jax.experimental.pallas import tpu as pltpu
from jax.experimental.pallas import tpu_sc as plsc

NUM_ROWS = 100000
DIM = 64
RANK = 8

NC = 2
NS = 16
NW = NC * NS
IDX_TOTAL = 204800
PER_W = IDX_TOTAL // NW
BLK_I = 128
STREAMS_PER_GROUP = 5
GROUP = STREAMS_PER_GROUP * BLK_I
NGROUPS = PER_W // GROUP

FUSE_BLK = 2048


def _fuse_body(wt_ref, bt_ref, a_ref, t2_ref):
    tt = wt_ref[...] + jnp.dot(a_ref[...].T, bt_ref[...],
                               preferred_element_type=jnp.float32)
    t = jnp.transpose(tt, (1, 0))
    t3 = t.reshape(FUSE_BLK // 2, 2, DIM)
    t2_ref[...] = jnp.concatenate([t3[:, 0, :], t3[:, 1, :]], axis=1)


def _fuse_table(Wt, A, Bt):
    grid = ((NUM_ROWS + FUSE_BLK - 1) // FUSE_BLK,)
    return pl.pallas_call(
        _fuse_body,
        grid=grid,
        in_specs=[
            pl.BlockSpec((DIM, FUSE_BLK), lambda i: (0, i)),
            pl.BlockSpec((RANK, FUSE_BLK), lambda i: (0, i)),
            pl.BlockSpec((RANK, DIM), lambda i: (0, 0)),
        ],
        out_specs=pl.BlockSpec((FUSE_BLK // 2, 2 * DIM), lambda i: (i, 0)),
        out_shape=jax.ShapeDtypeStruct((NUM_ROWS // 2, 2 * DIM), jnp.float32),
    )(Wt, Bt, A)


def _sc_gather(table, idx3):
    mesh = plsc.VectorSubcoreMesh(core_axis_name="c", subcore_axis_name="s")

    @functools.partial(
        pl.kernel,
        mesh=mesh,
        compiler_params=pltpu.CompilerParams(use_tc_tiling_on_sc=False),
        out_type=jax.ShapeDtypeStruct(
            (NW, NGROUPS, STREAMS_PER_GROUP, BLK_I, DIM), jnp.float32),
        scratch_types=[
            pltpu.VMEM((NGROUPS * STREAMS_PER_GROUP, BLK_I), jnp.int32),
            pltpu.VMEM((STREAMS_PER_GROUP, BLK_I, DIM), jnp.float32),
            pltpu.SemaphoreType.DMA,
        ],
    )
    def k(table_hbm, idx_hbm, out_hbm, idx_v, rows_v, gsem):
        wid = lax.axis_index("s") * NC + lax.axis_index("c")
        pltpu.sync_copy(idx_hbm.at[wid], idx_v)

        def body(g, carry):
            descs = []
            for s in range(STREAMS_PER_GROUP):
                descs.append(pltpu.async_copy(
                    table_hbm.at[idx_v.at[g * STREAMS_PER_GROUP + s]],
                    rows_v.at[s], gsem))
            for d in descs:
                d.wait()
            pltpu.sync_copy(rows_v, out_hbm.at[wid, g])
            return carry

        lax.fori_loop(0, NGROUPS, body, 0)

    return k(table, idx3)


def _xpose_body(g_ref, z_ref):
    gp = g_ref[0]
    g = jnp.stack([gp[:, :DIM], gp[:, DIM:]], axis=1).reshape(2048, DIM)
    z_ref[0] = jnp.transpose(g, (1, 0))


def _xpose(gp):
    return pl.pallas_call(
        _xpose_body,
        grid=(50, 2),
        in_specs=[pl.BlockSpec((1, 1024, 128), lambda j, kb: (j, kb, 0))],
        out_specs=pl.BlockSpec((1, DIM, 2048), lambda j, kb: (j, 0, kb)),
        out_shape=jax.ShapeDtypeStruct((50, DIM, 4096), jnp.float32),
    )(gp)


def kernel(x, W, A, B):
    t2 = _fuse_table(W.T, A, B.T)
    table = t2.reshape(NUM_ROWS, DIM)
    idx3 = x.T.reshape(NW, PER_W // BLK_I, BLK_I)
    out = _sc_gather(table, idx3)
    gp = out.reshape(50, 2048, 2 * DIM)
    z = _xpose(gp)
    return z.transpose(2, 0, 1)


# speedup vs baseline: 1.8363x; 1.3633x over previous
import functools

import jax
import jax.numpy as jnp
from jax import lax
from jax.experimental import pallas as pl
from jax.experimental.pallas import tpu as pltpu
from jax.experimental.pallas import tpu_sc as plsc

NUM_ROWS = 100000
DIM = 64
RANK = 8

NC = 2
NS = 16
NW = NC * NS
IDX_TOTAL = 204800
CHUNK = 512
NCHUNKS = IDX_TOTAL // CHUNK
ITERS = (NCHUNKS + NW - 1) // NW

FUSE_BLK = 2048
NBLK = (NUM_ROWS + FUSE_BLK - 1) // FUSE_BLK
NPAD = NBLK * FUSE_BLK


def _fuse_body(wt_ref, bt_ref, a_ref, t2_ref):
    tt = wt_ref[...] + jnp.dot(a_ref[...].T, bt_ref[...],
                               preferred_element_type=jnp.float32)
    u = jnp.concatenate([tt[:, :FUSE_BLK // 2], tt[:, FUSE_BLK // 2:]],
                        axis=0)
    t2_ref[...] = jnp.transpose(u, (1, 0))


def _fuse_table(Wt, A, Bt):
    return pl.pallas_call(
        _fuse_body,
        grid=(NBLK,),
        in_specs=[
            pl.BlockSpec((DIM, FUSE_BLK), lambda i: (0, i)),
            pl.BlockSpec((RANK, FUSE_BLK), lambda i: (0, i)),
            pl.BlockSpec((RANK, DIM), lambda i: (0, 0)),
        ],
        out_specs=pl.BlockSpec((FUSE_BLK // 2, 2 * DIM), lambda i: (i, 0)),
        out_shape=jax.ShapeDtypeStruct((NPAD // 2, 2 * DIM), jnp.float32),
    )(Wt, Bt, A)


def _sc_gather(table, idx):
    mesh = plsc.VectorSubcoreMesh(core_axis_name="c", subcore_axis_name="s")
    half = CHUNK // 2

    @functools.partial(
        pl.kernel,
        mesh=mesh,
        compiler_params=pltpu.CompilerParams(use_tc_tiling_on_sc=False),
        out_type=jax.ShapeDtypeStruct((NCHUNKS, half, 2 * DIM), jnp.float32),
        scratch_types=[
            pltpu.VMEM((2, half), jnp.int32),
            pltpu.VMEM((CHUNK, DIM), jnp.float32),
            pltpu.SemaphoreType.DMA,
        ],
    )
    def k(table_hbm, idx_hbm, out_hbm, raw_v, rows_v, gsem):
        wid = lax.axis_index("s") * NC + lax.axis_index("c")

        def body(it, carry):
            t = wid + NW * it

            @pl.when(t < NCHUNKS)
            def _():
                base = 2048 * (t // 4) + half * (t % 4)
                pltpu.sync_copy(idx_hbm.at[pl.ds(base, half)], raw_v.at[0])
                pltpu.sync_copy(idx_hbm.at[pl.ds(base + 1024, half)],
                                raw_v.at[1])
                descs = []
                for s in range(4):
                    descs.append(pltpu.async_copy(
                        table_hbm.at[raw_v.at[s // 2].at[pl.ds(128 * (s % 2), 128)]],
                        rows_v.at[pl.ds(128 * s, 128)], gsem))
                for d in descs:
                    d.wait()
                pltpu.sync_copy(rows_v.at[pl.ds(0, half)],
                                out_hbm.at[t].at[:, pl.ds(0, DIM)])
                pltpu.sync_copy(rows_v.at[pl.ds(half, half)],
                                out_hbm.at[t].at[:, pl.ds(DIM, DIM)])

            return carry

        lax.fori_loop(0, ITERS, body, 0)

    return k(table, idx)


def _xpose_body(g_ref, z_ref):
    gt = jnp.transpose(g_ref[0], (1, 0))
    z_ref[0] = jnp.concatenate([gt[:DIM], gt[DIM:]], axis=1)


def _xpose(gp):
    return pl.pallas_call(
        _xpose_body,
        grid=(50, 2),
        in_specs=[pl.BlockSpec((1, 1024, 128), lambda j, kb: (j, kb, 0))],
        out_specs=pl.BlockSpec((1, DIM, 2048), lambda j, kb: (j, 0, kb)),
        out_shape=jax.ShapeDtypeStruct((50, DIM, 4096), jnp.float32),
    )(gp)


def kernel(x, W, A, B):
    t2 = _fuse_table(W.T, A, B.T)
    table = t2.reshape(NPAD, DIM)
    xt = x.T.reshape(IDX_TOTAL)
    idx = (xt & ~jnp.int32(2047)) | ((xt & 1023) << 1) | ((xt >> 10) & 1)
    out = _sc_gather(table, idx)
    gp = out.reshape(50, 2048, 2 * DIM)
    z = _xpose(gp)
    return z.transpose(2, 0, 1)


# speedup vs baseline: 2.2200x; 1.2089x over previous
import functools

import jax
import jax.numpy as jnp
from jax import lax
from jax.experimental import pallas as pl
from jax.experimental.pallas import tpu as pltpu
from jax.experimental.pallas import tpu_sc as plsc

NUM_ROWS = 100000
DIM = 64
RANK = 8

NC = 2
NS = 16
NW = NC * NS
IDX_TOTAL = 204800
CHUNK = 512
NCHUNKS = IDX_TOTAL // CHUNK
ITERS = (NCHUNKS + NW - 1) // NW

FUSE_BLK = 4096
FUSE_H = FUSE_BLK // 2
NBLK = (NUM_ROWS + FUSE_BLK - 1) // FUSE_BLK
NPAD = NBLK * FUSE_BLK


def _fuse_body(wt_ref, bt_ref, a_ref, t2_ref):
    tt = wt_ref[...] + jnp.dot(a_ref[...].T, bt_ref[...],
                               preferred_element_type=jnp.float32)
    u = jnp.concatenate([tt[:, :FUSE_H], tt[:, FUSE_H:]],
                        axis=0)
    t2_ref[...] = jnp.transpose(u, (1, 0))


def _fuse_table(Wt, A, Bt):
    return pl.pallas_call(
        _fuse_body,
        grid=(NBLK,),
        in_specs=[
            pl.BlockSpec((DIM, FUSE_BLK), lambda i: (0, i)),
            pl.BlockSpec((RANK, FUSE_BLK), lambda i: (0, i)),
            pl.BlockSpec((RANK, DIM), lambda i: (0, 0)),
        ],
        out_specs=pl.BlockSpec((FUSE_BLK // 2, 2 * DIM), lambda i: (i, 0)),
        out_shape=jax.ShapeDtypeStruct((NPAD // 2, 2 * DIM), jnp.float32),
    )(Wt, Bt, A)


def _sc_gather(table, idx):
    mesh = plsc.VectorSubcoreMesh(core_axis_name="c", subcore_axis_name="s")
    half = CHUNK // 2

    @functools.partial(
        pl.kernel,
        mesh=mesh,
        compiler_params=pltpu.CompilerParams(use_tc_tiling_on_sc=False),
        out_type=jax.ShapeDtypeStruct((NCHUNKS, half, 2 * DIM), jnp.float32),
        scratch_types=[
            pltpu.VMEM((2, half), jnp.int32),
            pltpu.VMEM((CHUNK, DIM), jnp.float32),
            pltpu.SemaphoreType.DMA,
        ],
    )
    def k(table_hbm, idx_hbm, out_hbm, raw_v, rows_v, gsem):
        wid = lax.axis_index("s") * NC + lax.axis_index("c")

        def body(it, carry):
            t = wid + NW * it

            @pl.when(t < NCHUNKS)
            def _():
                base = 2048 * (t // 4) + half * (t % 4)
                pltpu.sync_copy(idx_hbm.at[pl.ds(base, half)], raw_v.at[0])
                pltpu.sync_copy(idx_hbm.at[pl.ds(base + 1024, half)],
                                raw_v.at[1])
                descs = []
                for s in range(4):
                    descs.append(pltpu.async_copy(
                        table_hbm.at[raw_v.at[s // 2].at[pl.ds(128 * (s % 2), 128)]],
                        rows_v.at[pl.ds(128 * s, 128)], gsem))
                for d in descs:
                    d.wait()
                pltpu.sync_copy(rows_v.at[pl.ds(0, half)],
                                out_hbm.at[t].at[:, pl.ds(0, DIM)])
                pltpu.sync_copy(rows_v.at[pl.ds(half, half)],
                                out_hbm.at[t].at[:, pl.ds(DIM, DIM)])

            return carry

        lax.fori_loop(0, ITERS, body, 0)

    return k(table, idx)


def _xpose_body(g_ref, z_ref):
    g = g_ref[0]
    gt0 = jnp.transpose(g[:1024], (1, 0))
    gt1 = jnp.transpose(g[1024:], (1, 0))
    z_ref[0] = jnp.concatenate(
        [gt0[:DIM], gt0[DIM:], gt1[:DIM], gt1[DIM:]], axis=1)


def _xpose(gp):
    return pl.pallas_call(
        _xpose_body,
        grid=(50,),
        in_specs=[pl.BlockSpec((1, 2048, 128), lambda j: (j, 0, 0))],
        out_specs=pl.BlockSpec((1, DIM, 4096), lambda j: (j, 0, 0)),
        out_shape=jax.ShapeDtypeStruct((50, DIM, 4096), jnp.float32),
    )(gp)


def kernel(x, W, A, B):
    t2 = _fuse_table(W.T, A, B.T)
    table = t2.reshape(NPAD, DIM)
    xt = x.T.reshape(IDX_TOTAL)
    idx = ((xt & ~jnp.int32(FUSE_BLK - 1)) | ((xt & (FUSE_H - 1)) << 1)
           | ((xt // FUSE_H) & 1))
    out = _sc_gather(table, idx)
    gp = out.reshape(50, 2048, 2 * DIM)
    z = _xpose(gp)
    return z.transpose(2, 0, 1)


# speedup vs baseline: 2.4211x; 1.0906x over previous
import functools

import jax
import jax.numpy as jnp
from jax import lax
from jax.experimental import pallas as pl
from jax.experimental.pallas import tpu as pltpu
from jax.experimental.pallas import tpu_sc as plsc

NUM_ROWS = 100000
DIM = 64
RANK = 8

NC = 2
NS = 16
NW = NC * NS
IDX_TOTAL = 204800
CHUNK = 512
NCHUNKS = IDX_TOTAL // CHUNK
ITERS = (NCHUNKS + NW - 1) // NW

FUSE_BLK = 4096
FUSE_H = FUSE_BLK // 2
NBLK = (NUM_ROWS + FUSE_BLK - 1) // FUSE_BLK
NPAD = NBLK * FUSE_BLK


def _fuse_body(wt_ref, bt_ref, a_ref, t2_ref):
    tt = wt_ref[...] + jnp.dot(a_ref[...].T, bt_ref[...],
                               preferred_element_type=jnp.float32)
    u = jnp.concatenate([tt[:, :FUSE_H], tt[:, FUSE_H:]],
                        axis=0)
    t2_ref[...] = jnp.transpose(u, (1, 0))


def _fuse_table(Wt, A, Bt):
    return pl.pallas_call(
        _fuse_body,
        grid=(NBLK,),
        in_specs=[
            pl.BlockSpec((DIM, FUSE_BLK), lambda i: (0, i)),
            pl.BlockSpec((RANK, FUSE_BLK), lambda i: (0, i)),
            pl.BlockSpec((RANK, DIM), lambda i: (0, 0)),
        ],
        out_specs=pl.BlockSpec((FUSE_BLK // 2, 2 * DIM), lambda i: (i, 0)),
        out_shape=jax.ShapeDtypeStruct((NPAD // 2, 2 * DIM), jnp.float32),
    )(Wt, Bt, A)


def _sc_gather(table, idx):
    mesh = plsc.VectorSubcoreMesh(core_axis_name="c", subcore_axis_name="s")
    half = CHUNK // 2

    @functools.partial(
        pl.kernel,
        mesh=mesh,
        compiler_params=pltpu.CompilerParams(use_tc_tiling_on_sc=False),
        out_type=jax.ShapeDtypeStruct((NCHUNKS, half, 2 * DIM), jnp.float32),
        scratch_types=[
            pltpu.VMEM((2 * CHUNK,), jnp.int32),
            pltpu.VMEM((2, CHUNK, DIM), jnp.float32),
            pltpu.SemaphoreType.DMA,
            pltpu.SemaphoreType.DMA,
        ],
    )
    def k(table_hbm, idx_hbm, out_hbm, raw_v, rows_v, gsem, wsem):
        wid = lax.axis_index("s") * NC + lax.axis_index("c")

        def chunk_of(it):
            return jnp.minimum(wid + NW * it, NCHUNKS - 1)

        def prep_idx(it):
            t = chunk_of(it)
            buf = lax.rem(it, 2)
            base = 2048 * (t // 4) + half * (t % 4)
            pltpu.sync_copy(idx_hbm.at[pl.ds(base, half)],
                            raw_v.at[pl.ds(CHUNK * buf, half)])
            pltpu.sync_copy(idx_hbm.at[pl.ds(base + 1024, half)],
                            raw_v.at[pl.ds(CHUNK * buf + half, half)])

        def fire(it):
            buf = lax.rem(it, 2)
            descs = []
            for s in range(4):
                descs.append(pltpu.async_copy(
                    table_hbm.at[raw_v.at[pl.ds(CHUNK * buf + 128 * s, 128)]],
                    rows_v.at[buf].at[pl.ds(128 * s, 128)], gsem))
            return descs

        def write(it):
            buf = lax.rem(it, 2)
            t = chunk_of(it)
            for h in range(2):
                pltpu.async_copy(
                    rows_v.at[buf].at[pl.ds(half * h, half)],
                    out_hbm.at[t].at[:, pl.ds(DIM * h, DIM)], wsem)

        def drain_write(it):
            buf = lax.rem(it, 2)
            for h in range(2):
                pltpu.make_async_copy(
                    table_hbm.at[pl.ds(0, half)],
                    rows_v.at[buf].at[pl.ds(half * h, half)], wsem).wait()

        prep_idx(0)
        d0 = fire(0)
        prep_idx(1)
        for d in d0:
            d.wait()
        write(0)

        def body(it, carry):
            descs = fire(it)
            prep_idx(it + 1)
            for d in descs:
                d.wait()
            drain_write(it - 1)
            write(it)
            return carry

        lax.fori_loop(1, ITERS - 1, body, 0)

        it = ITERS - 1
        descs = fire(it)
        for d in descs:
            d.wait()
        drain_write(it - 1)
        write(it)
        drain_write(it)

    return k(table, idx)


def _xpose_body(g_ref, z_ref):
    g = g_ref[0]
    gt0 = jnp.transpose(g[:1024], (1, 0))
    gt1 = jnp.transpose(g[1024:], (1, 0))
    z_ref[0] = jnp.concatenate(
        [gt0[:DIM], gt0[DIM:], gt1[:DIM], gt1[DIM:]], axis=1)


def _xpose(gp):
    return pl.pallas_call(
        _xpose_body,
        grid=(50,),
        in_specs=[pl.BlockSpec((1, 2048, 128), lambda j: (j, 0, 0))],
        out_specs=pl.BlockSpec((1, DIM, 4096), lambda j: (j, 0, 0)),
        out_shape=jax.ShapeDtypeStruct((50, DIM, 4096), jnp.float32),
    )(gp)


def kernel(x, W, A, B):
    t2 = _fuse_table(W.T, A, B.T)
    table = t2.reshape(NPAD, DIM)
    xt = x.T.reshape(IDX_TOTAL)
    idx = ((xt & ~jnp.int32(FUSE_BLK - 1)) | ((xt & (FUSE_H - 1)) << 1)
           | ((xt // FUSE_H) & 1))
    out = _sc_gather(table, idx)
    gp = out.reshape(50, 2048, 2 * DIM)
    z = _xpose(gp)
    return z.transpose(2, 0, 1)


# speedup vs baseline: 2.8539x; 1.1788x over previous
import functools

import jax
import jax.numpy as jnp
from jax import lax
from jax.experimental import pallas as pl
from jax.experimental.pallas import tpu as pltpu
from jax.experimental.pallas import tpu_sc as plsc

NUM_ROWS = 100000
DIM = 64
RANK = 8

NC = 2
NS = 16
NW = NC * NS
IDX_TOTAL = 204800
CHUNK = 512
NCHUNKS = IDX_TOTAL // CHUNK
ITERS = (NCHUNKS + NW - 1) // NW

FUSE_BLK = 8192
FUSE_H = FUSE_BLK // 2
NBLK = (NUM_ROWS + FUSE_BLK - 1) // FUSE_BLK
NPAD = NBLK * FUSE_BLK


def _fuse_body(wt_ref, bt_ref, a_ref, t2_ref):
    tt = wt_ref[...] + jnp.dot(a_ref[...].T, bt_ref[...],
                               preferred_element_type=jnp.float32)
    u = jnp.concatenate([tt[:, :FUSE_H], tt[:, FUSE_H:]],
                        axis=0)
    t2_ref[...] = jnp.transpose(u, (1, 0))


def _fuse_table(Wt, A, Bt):
    return pl.pallas_call(
        _fuse_body,
        grid=(NBLK,),
        in_specs=[
            pl.BlockSpec((DIM, FUSE_BLK), lambda i: (0, i)),
            pl.BlockSpec((RANK, FUSE_BLK), lambda i: (0, i)),
            pl.BlockSpec((RANK, DIM), lambda i: (0, 0)),
        ],
        out_specs=pl.BlockSpec((FUSE_BLK // 2, 2 * DIM), lambda i: (i, 0)),
        out_shape=jax.ShapeDtypeStruct((NPAD // 2, 2 * DIM), jnp.float32),
    )(Wt, Bt, A)


def _sc_gather(table, idx):
    mesh = plsc.VectorSubcoreMesh(core_axis_name="c", subcore_axis_name="s")
    half = CHUNK // 2

    @functools.partial(
        pl.kernel,
        mesh=mesh,
        compiler_params=pltpu.CompilerParams(use_tc_tiling_on_sc=False),
        out_type=jax.ShapeDtypeStruct((NCHUNKS, half, 2 * DIM), jnp.float32),
        scratch_types=[
            pltpu.VMEM((2 * CHUNK,), jnp.int32),
            pltpu.VMEM((2, CHUNK, DIM), jnp.float32),
            pltpu.SemaphoreType.DMA,
            pltpu.SemaphoreType.DMA,
        ],
    )
    def k(table_hbm, idx_hbm, out_hbm, raw_v, rows_v, gsem, wsem):
        wid = lax.axis_index("s") * NC + lax.axis_index("c")

        def chunk_of(it):
            return jnp.minimum(wid + NW * it, NCHUNKS - 1)

        def prep_idx(it):
            t = chunk_of(it)
            buf = lax.rem(it, 2)
            base = 2048 * (t // 4) + half * (t % 4)
            pltpu.sync_copy(idx_hbm.at[pl.ds(base, half)],
                            raw_v.at[pl.ds(CHUNK * buf, half)])
            pltpu.sync_copy(idx_hbm.at[pl.ds(base + 1024, half)],
                            raw_v.at[pl.ds(CHUNK * buf + half, half)])

        def fire(it):
            buf = lax.rem(it, 2)
            descs = []
            for s in range(4):
                descs.append(pltpu.async_copy(
                    table_hbm.at[raw_v.at[pl.ds(CHUNK * buf + 128 * s, 128)]],
                    rows_v.at[buf].at[pl.ds(128 * s, 128)], gsem))
            return descs

        def write(it):
            buf = lax.rem(it, 2)
            t = chunk_of(it)
            for h in range(2):
                pltpu.async_copy(
                    rows_v.at[buf].at[pl.ds(half * h, half)],
                    out_hbm.at[t].at[:, pl.ds(DIM * h, DIM)], wsem)

        def drain_write(it):
            buf = lax.rem(it, 2)
            for h in range(2):
                pltpu.make_async_copy(
                    table_hbm.at[pl.ds(0, half)],
                    rows_v.at[buf].at[pl.ds(half * h, half)], wsem).wait()

        prep_idx(0)
        d0 = fire(0)
        prep_idx(1)
        for d in d0:
            d.wait()
        write(0)

        def body(it, carry):
            descs = fire(it)
            prep_idx(it + 1)
            for d in descs:
                d.wait()
            drain_write(it - 1)
            write(it)
            return carry

        lax.fori_loop(1, ITERS - 1, body, 0)

        it = ITERS - 1
        descs = fire(it)
        for d in descs:
            d.wait()
        drain_write(it - 1)
        write(it)
        drain_write(it)

    return k(table, idx)


def _xpose_body(g_ref, z_ref):
    for r in range(2):
        g = g_ref[r]
        gt0 = jnp.transpose(g[:1024], (1, 0))
        gt1 = jnp.transpose(g[1024:], (1, 0))
        z_ref[r] = jnp.concatenate(
            [gt0[:DIM], gt0[DIM:], gt1[:DIM], gt1[DIM:]], axis=1)


def _xpose(gp):
    return pl.pallas_call(
        _xpose_body,
        grid=(25,),
        in_specs=[pl.BlockSpec((2, 2048, 128), lambda j: (j, 0, 0))],
        out_specs=pl.BlockSpec((2, DIM, 4096), lambda j: (j, 0, 0)),
        out_shape=jax.ShapeDtypeStruct((50, DIM, 4096), jnp.float32),
    )(gp)


def kernel(x, W, A, B):
    t2 = _fuse_table(W.T, A, B.T)
    table = t2.reshape(NPAD, DIM)
    xt = x.T.reshape(IDX_TOTAL)
    idx = ((xt & ~jnp.int32(FUSE_BLK - 1)) | ((xt & (FUSE_H - 1)) << 1)
           | ((xt // FUSE_H) & 1))
    out = _sc_gather(table, idx)
    gp = out.reshape(50, 2048, 2 * DIM)
    z = _xpose(gp)
    return z.transpose(2, 0, 1)


# speedup vs baseline: 3.0384x; 1.0646x over previous
import functools

import jax
import jax.numpy as jnp
from jax import lax
from jax.experimental import pallas as pl
from jax.experimental.pallas import tpu as pltpu
from jax.experimental.pallas import tpu_sc as plsc

NUM_ROWS = 100000
DIM = 64
RANK = 8

NC = 2
NS = 16
NW = NC * NS
IDX_TOTAL = 204800
CHUNK = 512
NCHUNKS = IDX_TOTAL // CHUNK
ITERS = (NCHUNKS + NW - 1) // NW

FUSE_BLK = 16384
FUSE_H = FUSE_BLK // 2
NBLK = (NUM_ROWS + FUSE_BLK - 1) // FUSE_BLK
NPAD = NBLK * FUSE_BLK


def _fuse_body(wt_ref, bt_ref, a_ref, t2_ref):
    tt = wt_ref[...] + jnp.dot(a_ref[...].T, bt_ref[...],
                               preferred_element_type=jnp.float32)
    u = jnp.concatenate([tt[:, :FUSE_H], tt[:, FUSE_H:]],
                        axis=0)
    t2_ref[...] = jnp.transpose(u, (1, 0))


def _fuse_table(Wt, A, Bt):
    return pl.pallas_call(
        _fuse_body,
        grid=(NBLK,),
        in_specs=[
            pl.BlockSpec((DIM, FUSE_BLK), lambda i: (0, i)),
            pl.BlockSpec((RANK, FUSE_BLK), lambda i: (0, i)),
            pl.BlockSpec((RANK, DIM), lambda i: (0, 0)),
        ],
        out_specs=pl.BlockSpec((FUSE_BLK // 2, 2 * DIM), lambda i: (i, 0)),
        out_shape=jax.ShapeDtypeStruct((NPAD // 2, 2 * DIM), jnp.float32),
    )(Wt, Bt, A)


def _sc_gather(table, idx):
    mesh = plsc.VectorSubcoreMesh(core_axis_name="c", subcore_axis_name="s")
    half = CHUNK // 2

    @functools.partial(
        pl.kernel,
        mesh=mesh,
        compiler_params=pltpu.CompilerParams(use_tc_tiling_on_sc=False),
        out_type=jax.ShapeDtypeStruct((NCHUNKS, half, 2 * DIM), jnp.float32),
        scratch_types=[
            pltpu.VMEM((2 * CHUNK,), jnp.int32),
            pltpu.VMEM((2, CHUNK, DIM), jnp.float32),
            pltpu.SemaphoreType.DMA,
            pltpu.SemaphoreType.DMA,
        ],
    )
    def k(table_hbm, idx_hbm, out_hbm, raw_v, rows_v, gsem, wsem):
        wid = lax.axis_index("s") * NC + lax.axis_index("c")

        def chunk_of(it):
            return jnp.minimum(wid + NW * it, NCHUNKS - 1)

        def prep_idx(it):
            t = chunk_of(it)
            buf = lax.rem(it, 2)
            base = 2048 * (t // 4) + half * (t % 4)
            pltpu.sync_copy(idx_hbm.at[pl.ds(base, half)],
                            raw_v.at[pl.ds(CHUNK * buf, half)])
            pltpu.sync_copy(idx_hbm.at[pl.ds(base + 1024, half)],
                            raw_v.at[pl.ds(CHUNK * buf + half, half)])

        def fire(it):
            buf = lax.rem(it, 2)
            descs = []
            for s in range(4):
                descs.append(pltpu.async_copy(
                    table_hbm.at[raw_v.at[pl.ds(CHUNK * buf + 128 * s, 128)]],
                    rows_v.at[buf].at[pl.ds(128 * s, 128)], gsem))
            return descs

        def write(it):
            buf = lax.rem(it, 2)
            t = chunk_of(it)
            for h in range(2):
                pltpu.async_copy(
                    rows_v.at[buf].at[pl.ds(half * h, half)],
                    out_hbm.at[t].at[:, pl.ds(DIM * h, DIM)], wsem)

        def drain_write(it):
            buf = lax.rem(it, 2)
            for h in range(2):
                pltpu.make_async_copy(
                    table_hbm.at[pl.ds(0, half)],
                    rows_v.at[buf].at[pl.ds(half * h, half)], wsem).wait()

        prep_idx(0)
        d0 = fire(0)
        prep_idx(1)
        for d in d0:
            d.wait()
        write(0)

        def body(it, carry):
            descs = fire(it)
            prep_idx(it + 1)
            for d in descs:
                d.wait()
            drain_write(it - 1)
            write(it)
            return carry

        lax.fori_loop(1, ITERS - 1, body, 0)

        it = ITERS - 1
        descs = fire(it)
        for d in descs:
            d.wait()
        drain_write(it - 1)
        write(it)
        drain_write(it)

    return k(table, idx)


def _xpose_body(g_ref, z_ref):
    for r in range(5):
        g = g_ref[r]
        gt0 = jnp.transpose(g[:1024], (1, 0))
        gt1 = jnp.transpose(g[1024:], (1, 0))
        z_ref[r] = jnp.concatenate(
            [gt0[:DIM], gt0[DIM:], gt1[:DIM], gt1[DIM:]], axis=1)


def _xpose(gp):
    return pl.pallas_call(
        _xpose_body,
        grid=(10,),
        in_specs=[pl.BlockSpec((5, 2048, 128), lambda j: (j, 0, 0))],
        out_specs=pl.BlockSpec((5, DIM, 4096), lambda j: (j, 0, 0)),
        out_shape=jax.ShapeDtypeStruct((50, DIM, 4096), jnp.float32),
    )(gp)


def kernel(x, W, A, B):
    t2 = _fuse_table(W.T, A, B.T)
    table = t2.reshape(NPAD, DIM)
    xt = x.T.reshape(IDX_TOTAL)
    idx = ((xt & ~jnp.int32(FUSE_BLK - 1)) | ((xt & (FUSE_H - 1)) << 1)
           | ((xt // FUSE_H) & 1))
    out = _sc_gather(table, idx)
    gp = out.reshape(50, 2048, 2 * DIM)
    z = _xpose(gp)
    return z.transpose(2, 0, 1)
